# baseline (speedup 1.0000x reference)
import functools

import jax
import jax.numpy as jnp
from jax.experimental import pallas as pl
from jax.experimental.pallas import tpu as pltpu

_G = 16


def _lm_kernel(x_ref, l_ref, pc_ref, br_ref, heat_ref, pts_ref, *, K, Hh, Wh):
    xv = x_ref[...]
    pcv = pc_ref[...]
    M = _G * Hh
    NG = (K // _G)
    rem = K - NG * _G

    def chain(off_k, n_k):
        Mn = n_k * Hh
        lg = l_ref[pl.ds(off_k * Hh, Mn), :]
        hr = jnp.dot(lg, xv, preferred_element_type=jnp.float32)
        hm = jnp.dot(hr, pcv, preferred_element_type=jnp.float32)
        hm = hm + br_ref[pl.ds(off_k * Hh, Mn), :]
        heat_ref[pl.ds(off_k, n_k), :, :] = hm.reshape(n_k, Hh, Wh)

    def body(g, _):
        chain(g * _G, _G)
        return 0

    jax.lax.fori_loop(0, NG, body, 0)
    if rem:
        chain(NG * _G, rem)

    hm = heat_ref[...]
    HW = Hh * Wh
    ih = jax.lax.broadcasted_iota(jnp.int32, (1, Hh, Wh), 1)
    iw = jax.lax.broadcasted_iota(jnp.int32, (1, Hh, Wh), 2)
    fi = ih * Wh + iw

    m = jnp.max(jnp.max(hm, axis=1, keepdims=True), axis=2, keepdims=True)
    cand = jnp.where(hm == m, fi, HW)
    idx = jnp.min(jnp.min(cand, axis=1, keepdims=True), axis=2, keepdims=True)

    log2w = Wh.bit_length() - 1
    px = idx & (Wh - 1)
    py = jnp.right_shift(idx, log2w)
    inb = (px > 0) & (px < Wh - 1) & (py > 0) & (py < Hh - 1)

    oh = (fi == idx).astype(jnp.float32)
    diff_x = pltpu.roll(hm, Wh - 1, 2) - pltpu.roll(hm, 1, 2)
    diff_y = pltpu.roll(hm, Hh - 1, 1) - pltpu.roll(hm, 1, 1)
    dx = jnp.sum(jnp.sum(oh * diff_x, axis=1, keepdims=True), axis=2, keepdims=True)
    dy = jnp.sum(jnp.sum(oh * diff_y, axis=1, keepdims=True), axis=2, keepdims=True)

    fx = px.astype(jnp.float32) + 1.0 + jnp.where(inb, jnp.sign(dx) * 0.25, 0.0) - 0.5
    fy = py.astype(jnp.float32) + 1.0 + jnp.where(inb, jnp.sign(dy) * 0.25, 0.0) - 0.5
    fx = (fx * 4.0 - 127.5) / 127.5
    fy = (fy * 4.0 - 127.5) / 127.5

    pts_ref[...] = jnp.concatenate([fx[:, :, 0], fy[:, :, 0]], axis=1)


def kernel(x, w, b):
    B, C, Hin, Win = x.shape
    Hh, Wh = Hin // 4, Win // 4
    K = w.shape[1]
    assert (Wh & (Wh - 1)) == 0, "flat-index math assumes Win//4 is a power of two"

    x2 = x.reshape(B, C * Hin, Win)

    rp = ((jnp.arange(Hh, dtype=jnp.int32)[:, None]
           == (jnp.arange(Hin, dtype=jnp.int32)[None, :] // 4))
          .astype(jnp.float32) * 0.25)
    pc = (((jnp.arange(Win, dtype=jnp.int32)[:, None] // 4)
           == jnp.arange(Wh, dtype=jnp.int32)[None, :])
          .astype(jnp.float32) * 0.25)
    wT = w.T
    L = (wT[:, None, :, None] * rp[None, :, None, :]).reshape(K * Hh, C * Hin)
    brep = jnp.repeat(b.reshape(K, 1), Hh, axis=0)

    heat, pts = pl.pallas_call(
        functools.partial(_lm_kernel, K=K, Hh=Hh, Wh=Wh),
        out_shape=(
            jax.ShapeDtypeStruct((B, K, Hh, Wh), jnp.float32),
            jax.ShapeDtypeStruct((B, K, 2), jnp.float32),
        ),
        grid=(B,),
        in_specs=[
            pl.BlockSpec((None, C * Hin, Win), lambda bi: (bi, 0, 0)),
            pl.BlockSpec((K * Hh, C * Hin), lambda bi: (0, 0)),
            pl.BlockSpec((Win, Wh), lambda bi: (0, 0)),
            pl.BlockSpec((K * Hh, 1), lambda bi: (0, 0)),
        ],
        out_specs=(
            pl.BlockSpec((None, K, Hh, Wh), lambda bi: (bi, 0, 0, 0)),
            pl.BlockSpec((None, K, 2), lambda bi: (bi, 0, 0)),
        ),
        compiler_params=pltpu.CompilerParams(
            dimension_semantics=("parallel",),
            vmem_limit_bytes=63 * 1024 * 1024,
        ),
    )(x2, L, pc, brep)

    return heat, pts


# speedup vs baseline: 1.0547x; 1.0547x over previous
import functools

import jax
import jax.numpy as jnp
from jax.experimental import pallas as pl
from jax.experimental.pallas import tpu as pltpu

_G = 16


def _lm_kernel(x_ref, l_ref, pc_ref, br_ref, heat_ref, pts_ref, *, K, Hh, Wh):
    xv = x_ref[...]
    pcv = pc_ref[...]
    M = _G * Hh
    NG = (K // _G)
    rem = K - NG * _G

    def chain(off_k, n_k):
        Mn = n_k * Hh
        lg = l_ref[pl.ds(off_k * Hh, Mn), :]
        hr = jnp.dot(lg, xv, preferred_element_type=jnp.float32)
        hm = jnp.dot(hr, pcv, preferred_element_type=jnp.float32)
        hm = hm + br_ref[pl.ds(off_k * Hh, Mn), :]
        heat_ref[pl.ds(off_k, n_k), :, :] = hm.reshape(n_k, Hh, Wh)

    def refine(koff, nk):
        hm = heat_ref[pl.ds(koff, nk), :, :]
        HW = Hh * Wh
        ih = jax.lax.broadcasted_iota(jnp.int32, (1, Hh, Wh), 1)
        iw = jax.lax.broadcasted_iota(jnp.int32, (1, Hh, Wh), 2)
        fi = ih * Wh + iw

        m = jnp.max(jnp.max(hm, axis=1, keepdims=True), axis=2, keepdims=True)
        cand = jnp.where(hm == m, fi, HW)
        idx = jnp.min(jnp.min(cand, axis=1, keepdims=True), axis=2, keepdims=True)

        log2w = Wh.bit_length() - 1
        px = idx & (Wh - 1)
        py = jnp.right_shift(idx, log2w)
        inb = (px > 0) & (px < Wh - 1) & (py > 0) & (py < Hh - 1)

        sel_x = (fi == idx + 1).astype(jnp.float32) - (fi == idx - 1).astype(jnp.float32)
        sel_y = (fi == idx + Wh).astype(jnp.float32) - (fi == idx - Wh).astype(jnp.float32)
        dx = jnp.sum(jnp.sum(hm * sel_x, axis=1, keepdims=True), axis=2, keepdims=True)
        dy = jnp.sum(jnp.sum(hm * sel_y, axis=1, keepdims=True), axis=2, keepdims=True)

        fx = px.astype(jnp.float32) + 1.0 + jnp.where(inb, jnp.sign(dx) * 0.25, 0.0) - 0.5
        fy = py.astype(jnp.float32) + 1.0 + jnp.where(inb, jnp.sign(dy) * 0.25, 0.0) - 0.5
        fx = (fx * 4.0 - 127.5) / 127.5
        fy = (fy * 4.0 - 127.5) / 127.5

        pts_ref[pl.ds(koff, nk), :] = jnp.concatenate(
            [fx[:, :, 0], fy[:, :, 0]], axis=1)

    chain(0, _G)

    def body(g, _):
        chain(g * _G, _G)
        refine((g - 1) * _G, _G)
        return 0

    jax.lax.fori_loop(1, NG, body, 0)
    if rem:
        chain(NG * _G, rem)
    refine((NG - 1) * _G, _G)
    if rem:
        refine(NG * _G, rem)


def kernel(x, w, b):
    B, C, Hin, Win = x.shape
    Hh, Wh = Hin // 4, Win // 4
    K = w.shape[1]
    assert (Wh & (Wh - 1)) == 0, "flat-index math assumes Win//4 is a power of two"

    x2 = x.reshape(B, C * Hin, Win)

    rp = ((jnp.arange(Hh, dtype=jnp.int32)[:, None]
           == (jnp.arange(Hin, dtype=jnp.int32)[None, :] // 4))
          .astype(jnp.float32) * 0.25)
    pc = (((jnp.arange(Win, dtype=jnp.int32)[:, None] // 4)
           == jnp.arange(Wh, dtype=jnp.int32)[None, :])
          .astype(jnp.float32) * 0.25)
    wT = w.T
    L = (wT[:, None, :, None] * rp[None, :, None, :]).reshape(K * Hh, C * Hin)
    brep = jnp.repeat(b.reshape(K, 1), Hh, axis=0)

    heat, pts = pl.pallas_call(
        functools.partial(_lm_kernel, K=K, Hh=Hh, Wh=Wh),
        out_shape=(
            jax.ShapeDtypeStruct((B, K, Hh, Wh), jnp.float32),
            jax.ShapeDtypeStruct((B, K, 2), jnp.float32),
        ),
        grid=(B,),
        in_specs=[
            pl.BlockSpec((None, C * Hin, Win), lambda bi: (bi, 0, 0)),
            pl.BlockSpec((K * Hh, C * Hin), lambda bi: (0, 0)),
            pl.BlockSpec((Win, Wh), lambda bi: (0, 0)),
            pl.BlockSpec((K * Hh, 1), lambda bi: (0, 0)),
        ],
        out_specs=(
            pl.BlockSpec((None, K, Hh, Wh), lambda bi: (bi, 0, 0, 0)),
            pl.BlockSpec((None, K, 2), lambda bi: (bi, 0, 0)),
        ),
        compiler_params=pltpu.CompilerParams(
            dimension_semantics=("parallel",),
            vmem_limit_bytes=63 * 1024 * 1024,
        ),
    )(x2, L, pc, brep)

    return heat, pts


# speedup vs baseline: 1.1473x; 1.0878x over previous
import functools

import jax
import jax.numpy as jnp
from jax.experimental import pallas as pl
from jax.experimental.pallas import tpu as pltpu

_G = 16


def _lm_kernel(x_ref, l_ref, pc_ref, br_ref, heat_ref, pts_ref, *, K, Hh, Wh):
    xv = x_ref[...]
    pcv = pc_ref[...]
    M = _G * Hh
    NG = (K // _G)
    rem = K - NG * _G

    def chain(off_k, n_k):
        Mn = n_k * Hh
        lg = l_ref[pl.ds(off_k * Hh, Mn), :]
        hr = jnp.dot(lg, xv, preferred_element_type=jnp.float32)
        hm = jnp.dot(hr, pcv, preferred_element_type=jnp.float32)
        hm = hm + br_ref[pl.ds(off_k * Hh, Mn), :]
        heat_ref[pl.ds(off_k, n_k), :, :] = hm.reshape(n_k, Hh, Wh)

    def refine(koff, nk):
        hm = heat_ref[pl.ds(koff, nk), :, :]
        HW = Hh * Wh
        ih = jax.lax.broadcasted_iota(jnp.int32, (1, Hh, Wh), 1)
        iw = jax.lax.broadcasted_iota(jnp.int32, (1, Hh, Wh), 2)
        fi = ih * Wh + iw

        m = jnp.max(jnp.max(hm, axis=1, keepdims=True), axis=2, keepdims=True)
        cand = jnp.where(hm == m, fi, HW)
        idx = jnp.min(jnp.min(cand, axis=1, keepdims=True), axis=2, keepdims=True)

        log2w = Wh.bit_length() - 1
        px = idx & (Wh - 1)
        py = jnp.right_shift(idx, log2w)
        inb = (px > 0) & (px < Wh - 1) & (py > 0) & (py < Hh - 1)

        idxb = jnp.broadcast_to(idx, hm.shape)
        sel_x = (fi - 1 == idxb).astype(jnp.float32) - (fi + 1 == idxb).astype(jnp.float32)
        sel_y = (fi - Wh == idxb).astype(jnp.float32) - (fi + Wh == idxb).astype(jnp.float32)
        dx = jnp.sum(jnp.sum(hm * sel_x, axis=1, keepdims=True), axis=2, keepdims=True)
        dy = jnp.sum(jnp.sum(hm * sel_y, axis=1, keepdims=True), axis=2, keepdims=True)

        fx = px.astype(jnp.float32) + 1.0 + jnp.where(inb, jnp.sign(dx) * 0.25, 0.0) - 0.5
        fy = py.astype(jnp.float32) + 1.0 + jnp.where(inb, jnp.sign(dy) * 0.25, 0.0) - 0.5
        fx = (fx * 4.0 - 127.5) / 127.5
        fy = (fy * 4.0 - 127.5) / 127.5

        pts_ref[pl.ds(koff, nk), :] = jnp.concatenate(
            [fx[:, :, 0], fy[:, :, 0]], axis=1)

    def body(g, _):
        chain(g * _G, _G)
        return 0

    jax.lax.fori_loop(0, NG, body, 0)
    if rem:
        chain(NG * _G, rem)
    refine(0, K)


def kernel(x, w, b):
    B, C, Hin, Win = x.shape
    Hh, Wh = Hin // 4, Win // 4
    K = w.shape[1]
    assert (Wh & (Wh - 1)) == 0, "flat-index math assumes Win//4 is a power of two"

    x2 = x.reshape(B, C * Hin, Win)

    rp = ((jnp.arange(Hh, dtype=jnp.int32)[:, None]
           == (jnp.arange(Hin, dtype=jnp.int32)[None, :] // 4))
          .astype(jnp.float32) * 0.25)
    pc = (((jnp.arange(Win, dtype=jnp.int32)[:, None] // 4)
           == jnp.arange(Wh, dtype=jnp.int32)[None, :])
          .astype(jnp.float32) * 0.25)
    wT = w.T
    L = (wT[:, None, :, None] * rp[None, :, None, :]).reshape(K * Hh, C * Hin)
    brep = jnp.repeat(b.reshape(K, 1), Hh, axis=0)

    heat, pts = pl.pallas_call(
        functools.partial(_lm_kernel, K=K, Hh=Hh, Wh=Wh),
        out_shape=(
            jax.ShapeDtypeStruct((B, K, Hh, Wh), jnp.float32),
            jax.ShapeDtypeStruct((B, K, 2), jnp.float32),
        ),
        grid=(B,),
        in_specs=[
            pl.BlockSpec((None, C * Hin, Win), lambda bi: (bi, 0, 0)),
            pl.BlockSpec((K * Hh, C * Hin), lambda bi: (0, 0)),
            pl.BlockSpec((Win, Wh), lambda bi: (0, 0)),
            pl.BlockSpec((K * Hh, 1), lambda bi: (0, 0)),
        ],
        out_specs=(
            pl.BlockSpec((None, K, Hh, Wh), lambda bi: (bi, 0, 0, 0)),
            pl.BlockSpec((None, K, 2), lambda bi: (bi, 0, 0)),
        ),
        compiler_params=pltpu.CompilerParams(
            dimension_semantics=("parallel",),
            vmem_limit_bytes=63 * 1024 * 1024,
        ),
    )(x2, L, pc, brep)

    return heat, pts
